# baseline (speedup 1.0000x reference)
import functools

import jax
import jax.numpy as jnp
from jax import lax
from jax.experimental import pallas as pl
from jax.experimental.pallas import tpu as pltpu
from jax.experimental.pallas import tpu_sc as plsc

_N = 1_000_000
_TABLE = 100
_LANES = 16

_info = plsc.get_sparse_core_info()
_NC = _info.num_cores
_NS = _info.num_subcores
_NW = _NC * _NS

_C0 = 31232 * _LANES
_C1 = _N - _C0


def _make_gather(n_el, z_base):
    nvec = n_el // _LANES
    vpw = nvec // _NW
    per_w = vpw * _LANES
    rem_v = nvec - vpw * _NW
    rem_base = per_w * _NW
    va = (vpw // 2) & ~7
    vb_full = ((vpw - va) // 8) * 8
    ea = va * _LANES

    @functools.partial(
        pl.kernel,
        out_type=jax.ShapeDtypeStruct((n_el,), jnp.float32),
        mesh=plsc.VectorSubcoreMesh(core_axis_name="c", subcore_axis_name="s"),
        compiler_params=pltpu.CompilerParams(needs_layout_passes=False),
        scratch_types=[
            pltpu.VMEM((_TABLE,), jnp.float32),
            pltpu.VMEM((per_w + _LANES,), jnp.int32),
            pltpu.VMEM((per_w + _LANES,), jnp.float32),
            pltpu.SemaphoreType.DMA,
            pltpu.SemaphoreType.DMA,
            pltpu.SemaphoreType.DMA,
            pltpu.SemaphoreType.DMA,
        ],
    )
    def gather(z_hbm, tab_hbm, out_hbm, tab_v, z_v, o_v,
               sem_a, sem_b, sem_oa, sem_ob):
        wid = lax.axis_index("s") * _NC + lax.axis_index("c")
        base = wid * per_w
        rem_off = rem_base + wid * _LANES

        cp_a = pltpu.async_copy(z_hbm.at[pl.ds(z_base + base, ea)],
                                z_v.at[pl.ds(0, ea)], sem_a)
        cp_b = pltpu.async_copy(
            z_hbm.at[pl.ds(z_base + base + ea, per_w - ea)],
            z_v.at[pl.ds(ea, per_w - ea)], sem_b)
        pltpu.sync_copy(tab_hbm, tab_v)

        if rem_v:
            @pl.when(wid < rem_v)
            def _load_extra():
                pltpu.sync_copy(z_hbm.at[pl.ds(z_base + rem_off, _LANES)],
                                z_v.at[pl.ds(per_w, _LANES)])

        def one_vec(off):
            zv = z_v[pl.ds(off, _LANES)]
            o_v[pl.ds(off, _LANES)] = plsc.load_gather(tab_v, [zv])

        cp_a.wait()

        @plsc.parallel_loop(0, ea, _LANES, unroll=8)
        def _half_a(off):
            one_vec(off)

        out_a = pltpu.async_copy(o_v.at[pl.ds(0, ea)],
                                 out_hbm.at[pl.ds(base, ea)], sem_oa)
        cp_b.wait()

        @plsc.parallel_loop(ea, ea + vb_full * _LANES, _LANES, unroll=8)
        def _half_b(off):
            one_vec(off)

        for v in range(va + vb_full, vpw):
            one_vec(v * _LANES)

        if rem_v:
            @pl.when(wid < rem_v)
            def _do_extra():
                one_vec(per_w)

        out_b = pltpu.async_copy(o_v.at[pl.ds(ea, per_w - ea)],
                                 out_hbm.at[pl.ds(base + ea, per_w - ea)],
                                 sem_ob)

        if rem_v:
            @pl.when(wid < rem_v)
            def _store_extra():
                pltpu.sync_copy(o_v.at[pl.ds(per_w, _LANES)],
                                out_hbm.at[pl.ds(rem_off, _LANES)])

        out_a.wait()
        out_b.wait()

    return gather


_gather_c0 = _make_gather(_C0, 0)
_gather_c1 = _make_gather(_C1, _C0)


def kernel(x, z, atomref_weight):
    zi = jnp.ravel(z).astype(jnp.int32)
    tab = jnp.ravel(atomref_weight).astype(jnp.float32)
    xf = x.astype(jnp.float32)
    ref0 = _gather_c0(zi, tab)
    ref1 = _gather_c1(zi, tab)
    out0 = lax.slice(xf, (0, 0), (_C0, 1)) + ref0.reshape(_C0, 1)
    out1 = lax.slice(xf, (_C0, 0), (_N, 1)) + ref1.reshape(_C1, 1)
    return lax.concatenate((out0, out1), 0)


# speedup vs baseline: 1.6488x; 1.6488x over previous
import functools

import jax
import jax.numpy as jnp
from jax import lax
from jax.experimental import pallas as pl
from jax.experimental.pallas import tpu as pltpu
from jax.experimental.pallas import tpu_sc as plsc

_N = 1_000_000
_TABLE = 100
_LANES = 16

_info = plsc.get_sparse_core_info()
_NC = _info.num_cores
_NS = _info.num_subcores
_NW = _NC * _NS

_NVEC = _N // _LANES
_VPW = _NVEC // _NW
_PER_W = _VPW * _LANES
_REM_V = _NVEC - _VPW * _NW
_REM_BASE = _PER_W * _NW

_VA = 976
_VB = _VPW - _VA
_EA = _VA * _LANES
_EB = _VB * _LANES


@functools.partial(
    pl.kernel,
    out_type=jax.ShapeDtypeStruct((_N,), jnp.float32),
    mesh=plsc.VectorSubcoreMesh(core_axis_name="c", subcore_axis_name="s"),
    compiler_params=pltpu.CompilerParams(needs_layout_passes=False),
    scratch_types=[
        pltpu.VMEM((_TABLE,), jnp.float32),
        pltpu.VMEM((_PER_W + _LANES,), jnp.int32),
        pltpu.VMEM((_PER_W + _LANES,), jnp.float32),
        pltpu.SemaphoreType.DMA,
        pltpu.SemaphoreType.DMA,
        pltpu.SemaphoreType.DMA,
        pltpu.SemaphoreType.DMA,
    ],
)
def _gather_sc(z_hbm, tab_hbm, out_hbm, tab_v, z_v, o_v,
               sem_a, sem_b, sem_oa, sem_ob):
    wid = lax.axis_index("s") * _NC + lax.axis_index("c")
    base = wid * _PER_W
    rem_off = _REM_BASE + wid * _LANES

    cp_a = pltpu.async_copy(z_hbm.at[pl.ds(base, _EA)],
                            z_v.at[pl.ds(0, _EA)], sem_a)
    cp_b = pltpu.async_copy(z_hbm.at[pl.ds(base + _EA, _EB)],
                            z_v.at[pl.ds(_EA, _EB)], sem_b)
    pltpu.sync_copy(tab_hbm, tab_v)

    @pl.when(wid < _REM_V)
    def _load_extra():
        pltpu.sync_copy(z_hbm.at[pl.ds(rem_off, _LANES)],
                        z_v.at[pl.ds(_PER_W, _LANES)])

    def one_vec(off):
        zv = z_v[pl.ds(off, _LANES)]
        o_v[pl.ds(off, _LANES)] = plsc.load_gather(tab_v, [zv])

    cp_a.wait()

    @plsc.parallel_loop(0, _EA, _LANES, unroll=8)
    def _half_a(off):
        one_vec(off)

    out_a = pltpu.async_copy(o_v.at[pl.ds(0, _EA)],
                             out_hbm.at[pl.ds(base, _EA)], sem_oa)
    cp_b.wait()

    @plsc.parallel_loop(_EA, _EA + _EB - _LANES, _LANES, unroll=8)
    def _half_b(off):
        one_vec(off)

    one_vec(_EA + _EB - _LANES)

    @pl.when(wid < _REM_V)
    def _do_extra():
        one_vec(_PER_W)

    out_b = pltpu.async_copy(o_v.at[pl.ds(_EA, _EB)],
                             out_hbm.at[pl.ds(base + _EA, _EB)], sem_ob)

    @pl.when(wid < _REM_V)
    def _store_extra():
        pltpu.sync_copy(o_v.at[pl.ds(_PER_W, _LANES)],
                        out_hbm.at[pl.ds(rem_off, _LANES)])

    out_a.wait()
    out_b.wait()


def kernel(x, z, atomref_weight):
    zi = jnp.ravel(z).astype(jnp.int32)
    tab = jnp.ravel(atomref_weight).astype(jnp.float32)
    ref1d = _gather_sc(zi, tab)
    return x.astype(jnp.float32) + ref1d.reshape(_N, 1)


# speedup vs baseline: 1.6505x; 1.0010x over previous
import functools

import jax
import jax.numpy as jnp
from jax import lax
from jax.experimental import pallas as pl
from jax.experimental.pallas import tpu as pltpu
from jax.experimental.pallas import tpu_sc as plsc

_N = 1_000_000
_TABLE = 100
_LANES = 16

_info = plsc.get_sparse_core_info()
_NC = _info.num_cores
_NS = _info.num_subcores
_NW = _NC * _NS

_NVEC = _N // _LANES
_VPW = _NVEC // _NW
_PER_W = _VPW * _LANES
_REM_V = _NVEC - _VPW * _NW
_REM_BASE = _PER_W * _NW

_VA = 976
_VB = _VPW - _VA
_EA = _VA * _LANES
_EB = _VB * _LANES


@functools.partial(
    pl.kernel,
    out_type=jax.ShapeDtypeStruct((_N,), jnp.float32),
    mesh=plsc.VectorSubcoreMesh(core_axis_name="c", subcore_axis_name="s"),
    compiler_params=pltpu.CompilerParams(needs_layout_passes=False),
    scratch_types=[
        pltpu.VMEM((_TABLE,), jnp.float32),
        pltpu.VMEM((_PER_W + _LANES,), jnp.int32),
        pltpu.VMEM((_PER_W + _LANES,), jnp.float32),
        pltpu.SemaphoreType.DMA,
        pltpu.SemaphoreType.DMA,
        pltpu.SemaphoreType.DMA,
        pltpu.SemaphoreType.DMA,
    ],
)
def _gather_sc(z_hbm, tab_hbm, out_hbm, tab_v, z_v, o_v,
               sem_a, sem_b, sem_oa, sem_ob):
    wid = lax.axis_index("s") * _NC + lax.axis_index("c")
    base = wid * _PER_W
    rem_off = _REM_BASE + wid * _LANES

    cp_a = pltpu.async_copy(z_hbm.at[pl.ds(base, _EA)],
                            z_v.at[pl.ds(0, _EA)], sem_a)
    cp_b = pltpu.async_copy(z_hbm.at[pl.ds(base + _EA, _EB)],
                            z_v.at[pl.ds(_EA, _EB)], sem_b)
    pltpu.sync_copy(tab_hbm, tab_v)

    def one_vec(off):
        zv = z_v[pl.ds(off, _LANES)]
        o_v[pl.ds(off, _LANES)] = plsc.load_gather(tab_v, [zv])

    cp_a.wait()

    @plsc.parallel_loop(0, _EA, _LANES, unroll=4)
    def _half_a(off):
        one_vec(off)

    out_a = pltpu.async_copy(o_v.at[pl.ds(0, _EA)],
                             out_hbm.at[pl.ds(base, _EA)], sem_oa)
    cp_b.wait()

    @plsc.parallel_loop(_EA, _EA + _EB - _LANES, _LANES, unroll=4)
    def _half_b(off):
        one_vec(off)

    one_vec(_EA + _EB - _LANES)

    out_b = pltpu.async_copy(o_v.at[pl.ds(_EA, _EB)],
                             out_hbm.at[pl.ds(base + _EA, _EB)], sem_ob)

    @pl.when(wid < _REM_V)
    def _extra():
        pltpu.sync_copy(z_hbm.at[pl.ds(rem_off, _LANES)],
                        z_v.at[pl.ds(_PER_W, _LANES)])
        one_vec(_PER_W)
        pltpu.sync_copy(o_v.at[pl.ds(_PER_W, _LANES)],
                        out_hbm.at[pl.ds(rem_off, _LANES)])

    out_a.wait()
    out_b.wait()


def kernel(x, z, atomref_weight):
    zi = jnp.ravel(z).astype(jnp.int32)
    tab = jnp.ravel(atomref_weight).astype(jnp.float32)
    ref1d = _gather_sc(zi, tab)
    return x.astype(jnp.float32) + ref1d.reshape(_N, 1)
